# Initial kernel scaffold; baseline (speedup 1.0000x reference)
import jax
import jax.numpy as jnp
from jax.experimental import pallas as pl


def kernel(query_indices, candidate_indices, q_table, c_table, q_w0, q_b0, q_w1, q_b1, q_w2, q_b2, c_w0, c_b0, c_w1, c_b1, c_w2, c_b2):
    raise NotImplementedError("write your pallas kernel here")


# speedup vs baseline: 14.1259x; 14.1259x over previous
import functools

import jax
import jax.numpy as jnp
from jax import lax
from jax.experimental import pallas as pl
from jax.experimental.pallas import tpu as pltpu
from jax.experimental.pallas import tpu_sc as plsc

_NC = 2
_NS = 16
_NW = _NC * _NS

_CW = 128


def _pool_sc(q_idx4, c_idx4, q_table, c_table, *, B, L, D):
    rows_per_w = B // _NW
    nk = rows_per_w // _CW

    mesh = plsc.VectorSubcoreMesh(core_axis_name="c", subcore_axis_name="s")

    @functools.partial(
        pl.kernel,
        mesh=mesh,
        out_type=(
            jax.ShapeDtypeStruct((B, D), jnp.float32),
            jax.ShapeDtypeStruct((B, D), jnp.float32),
        ),
        scratch_types=[
            pltpu.VMEM((L, nk, _CW), jnp.int32),
            pltpu.VMEM((rows_per_w, D), jnp.float32),
            pltpu.SemaphoreType.DMA,
        ],
    )
    def pool(q_idx_hbm, c_idx_hbm, q_tab_hbm, c_tab_hbm,
             q_out_hbm, c_out_hbm, idx_v, acc_v, sem):
        wid = lax.axis_index("s") * _NC + lax.axis_index("c")
        base = wid * rows_per_w

        def one_tower(idx_hbm, tab_hbm, out_hbm):
            pltpu.sync_copy(idx_hbm.at[wid], idx_v)
            cps = [
                pltpu.async_copy(
                    tab_hbm.at[idx_v.at[0, k]],
                    acc_v.at[pl.ds(k * _CW, _CW)],
                    sem,
                )
                for k in range(nk)
            ]
            for cp in cps:
                cp.wait()

            def passes(j, carry):
                cps = [
                    pltpu.async_copy(
                        tab_hbm.at[idx_v.at[j, k]],
                        acc_v.at[pl.ds(k * _CW, _CW)],
                        sem,
                        add=True,
                    )
                    for k in range(nk)
                ]
                for cp in cps:
                    cp.wait()
                return carry

            lax.fori_loop(1, L, passes, 0)
            pltpu.sync_copy(acc_v, out_hbm.at[pl.ds(base, rows_per_w)])

        one_tower(q_idx_hbm, q_tab_hbm, q_out_hbm)
        one_tower(c_idx_hbm, c_tab_hbm, c_out_hbm)

    return pool(q_idx4, c_idx4, q_table, c_table)


def _mlp_tc(q_pooled, c_pooled, q_ws, q_bs, c_ws, c_bs, *, B, D):
    blk = 2048
    grid = (B // blk,)

    n_layers = len(q_ws)
    out_d = q_ws[-1].shape[0]

    def body(qp_ref, cp_ref, *refs):
        q_wrefs = refs[0:n_layers]
        q_brefs = refs[n_layers:2 * n_layers]
        c_wrefs = refs[2 * n_layers:3 * n_layers]
        c_brefs = refs[3 * n_layers:4 * n_layers]
        q_out_ref, c_out_ref = refs[4 * n_layers:]

        def tower(x, wrefs, brefs):
            for w_ref, b_ref in zip(wrefs, brefs):
                y = lax.dot_general(
                    x, w_ref[...], (((1,), (1,)), ((), ())),
                    preferred_element_type=jnp.float32,
                )
                x = jnp.maximum(y + b_ref[...], 0.0)
            return x

        q_out_ref[...] = tower(qp_ref[...], q_wrefs, q_brefs)
        c_out_ref[...] = tower(cp_ref[...], c_wrefs, c_brefs)

    x_spec = pl.BlockSpec((blk, D), lambda i: (i, 0))
    full = lambda a: pl.BlockSpec(a.shape, lambda i: (0,) * a.ndim)
    in_specs = (
        [x_spec, x_spec]
        + [full(w) for w in q_ws] + [full(b) for b in q_bs]
        + [full(w) for w in c_ws] + [full(b) for b in c_bs]
    )
    out_specs = (
        pl.BlockSpec((blk, out_d), lambda i: (i, 0)),
        pl.BlockSpec((blk, out_d), lambda i: (i, 0)),
    )
    return pl.pallas_call(
        body,
        grid=grid,
        in_specs=in_specs,
        out_specs=out_specs,
        out_shape=(
            jax.ShapeDtypeStruct((B, out_d), jnp.float32),
            jax.ShapeDtypeStruct((B, out_d), jnp.float32),
        ),
    )(q_pooled, c_pooled, *q_ws, *q_bs, *c_ws, *c_bs)


def kernel(query_indices, candidate_indices, q_table, c_table,
           q_w0, q_b0, q_w1, q_b1, q_w2, q_b2,
           c_w0, c_b0, c_w1, c_b1, c_w2, c_b2):
    B, L = query_indices.shape
    V, D = q_table.shape
    rows_per_w = B // _NW
    nk = rows_per_w // _CW

    def prep(idx):
        idx = idx.astype(jnp.int32)
        return idx.reshape(_NW, nk, _CW, L).transpose(0, 3, 1, 2)

    q_pooled, c_pooled = _pool_sc(
        prep(query_indices), prep(candidate_indices), q_table, c_table,
        B=B, L=L, D=D,
    )

    q_bs = [b.reshape(1, -1) for b in (q_b0, q_b1, q_b2)]
    c_bs = [b.reshape(1, -1) for b in (c_b0, c_b1, c_b2)]
    return _mlp_tc(
        q_pooled, c_pooled,
        [q_w0, q_w1, q_w2], q_bs, [c_w0, c_w1, c_w2], c_bs,
        B=B, D=D,
    )


# speedup vs baseline: 15.7797x; 1.1171x over previous
import functools

import jax
import jax.numpy as jnp
from jax import lax
from jax.experimental import pallas as pl
from jax.experimental.pallas import tpu as pltpu
from jax.experimental.pallas import tpu_sc as plsc

_NC = 2
_NS = 16
_NW = _NC * _NS

_CW = 128


def _pool_sc(q_idx4, c_idx4, q_table, c_table, *, B, L, D):
    rows_per_w = B // _NW
    nk = rows_per_w // _CW

    mesh = plsc.VectorSubcoreMesh(core_axis_name="c", subcore_axis_name="s")

    @functools.partial(
        pl.kernel,
        mesh=mesh,
        out_type=(
            jax.ShapeDtypeStruct((B, D), jnp.float32),
            jax.ShapeDtypeStruct((B, D), jnp.float32),
        ),
        scratch_types=[
            pltpu.VMEM((L, nk, _CW), jnp.int32),
            pltpu.VMEM((rows_per_w, D), jnp.float32),
            pltpu.SemaphoreType.DMA,
        ],
    )
    def pool(q_idx_hbm, c_idx_hbm, q_tab_hbm, c_tab_hbm,
             q_out_hbm, c_out_hbm, idx_v, acc_v, sem):
        wid = lax.axis_index("s") * _NC + lax.axis_index("c")
        base = wid * rows_per_w

        def one_tower(idx_hbm, tab_hbm, out_hbm):
            pltpu.sync_copy(idx_hbm.at[wid], idx_v)
            cps = [
                pltpu.async_copy(
                    tab_hbm.at[idx_v.at[0, k]],
                    acc_v.at[pl.ds(k * _CW, _CW)],
                    sem,
                )
                for k in range(nk)
            ]
            for cp in cps:
                cp.wait()

            def passes(j, carry):
                for k in range(nk):
                    pltpu.async_copy(
                        tab_hbm.at[idx_v.at[j, k]],
                        acc_v.at[pl.ds(k * _CW, _CW)],
                        sem,
                        add=True,
                    )
                return carry

            lax.fori_loop(1, L, passes, 0)

            def drain(j, carry):
                for k in range(nk):
                    pltpu.make_async_copy(
                        tab_hbm.at[pl.ds(0, _CW)],
                        acc_v.at[pl.ds(k * _CW, _CW)],
                        sem,
                    ).wait()
                return carry

            lax.fori_loop(1, L, drain, 0)
            pltpu.sync_copy(acc_v, out_hbm.at[pl.ds(base, rows_per_w)])

        one_tower(q_idx_hbm, q_tab_hbm, q_out_hbm)
        one_tower(c_idx_hbm, c_tab_hbm, c_out_hbm)

    return pool(q_idx4, c_idx4, q_table, c_table)


def _mlp_tc(q_pooled, c_pooled, q_ws, q_bs, c_ws, c_bs, *, B, D):
    blk = 2048
    grid = (B // blk,)

    n_layers = len(q_ws)
    out_d = q_ws[-1].shape[0]

    def body(qp_ref, cp_ref, *refs):
        q_wrefs = refs[0:n_layers]
        q_brefs = refs[n_layers:2 * n_layers]
        c_wrefs = refs[2 * n_layers:3 * n_layers]
        c_brefs = refs[3 * n_layers:4 * n_layers]
        q_out_ref, c_out_ref = refs[4 * n_layers:]

        def tower(x, wrefs, brefs):
            for w_ref, b_ref in zip(wrefs, brefs):
                y = lax.dot_general(
                    x, w_ref[...], (((1,), (1,)), ((), ())),
                    preferred_element_type=jnp.float32,
                )
                x = jnp.maximum(y + b_ref[...], 0.0)
            return x

        q_out_ref[...] = tower(qp_ref[...], q_wrefs, q_brefs)
        c_out_ref[...] = tower(cp_ref[...], c_wrefs, c_brefs)

    x_spec = pl.BlockSpec((blk, D), lambda i: (i, 0))
    full = lambda a: pl.BlockSpec(a.shape, lambda i: (0,) * a.ndim)
    in_specs = (
        [x_spec, x_spec]
        + [full(w) for w in q_ws] + [full(b) for b in q_bs]
        + [full(w) for w in c_ws] + [full(b) for b in c_bs]
    )
    out_specs = (
        pl.BlockSpec((blk, out_d), lambda i: (i, 0)),
        pl.BlockSpec((blk, out_d), lambda i: (i, 0)),
    )
    return pl.pallas_call(
        body,
        grid=grid,
        in_specs=in_specs,
        out_specs=out_specs,
        out_shape=(
            jax.ShapeDtypeStruct((B, out_d), jnp.float32),
            jax.ShapeDtypeStruct((B, out_d), jnp.float32),
        ),
    )(q_pooled, c_pooled, *q_ws, *q_bs, *c_ws, *c_bs)


def kernel(query_indices, candidate_indices, q_table, c_table,
           q_w0, q_b0, q_w1, q_b1, q_w2, q_b2,
           c_w0, c_b0, c_w1, c_b1, c_w2, c_b2):
    B, L = query_indices.shape
    V, D = q_table.shape
    rows_per_w = B // _NW
    nk = rows_per_w // _CW

    def prep(idx):
        idx = idx.astype(jnp.int32)
        return idx.reshape(_NW, nk, _CW, L).transpose(0, 3, 1, 2)

    q_pooled, c_pooled = _pool_sc(
        prep(query_indices), prep(candidate_indices), q_table, c_table,
        B=B, L=L, D=D,
    )

    q_bs = [b.reshape(1, -1) for b in (q_b0, q_b1, q_b2)]
    c_bs = [b.reshape(1, -1) for b in (c_b0, c_b1, c_b2)]
    return _mlp_tc(
        q_pooled, c_pooled,
        [q_w0, q_w1, q_w2], q_bs, [c_w0, c_w1, c_w2], c_bs,
        B=B, D=D,
    )
